# baseline (speedup 1.0000x reference)
import functools
import math

import jax
import jax.numpy as jnp
from jax import lax
from jax.experimental import pallas as pl
from jax.experimental.pallas import tpu as pltpu
from jax.experimental.pallas import tpu_sc as plsc

N = 10000
E = 320000
D = 128
L = 8
ALPHA = 0.1
THETA = 0.5

NC = 2
NS = 16
NW = NC * NS
K = 128
SPT = 320
NPAD = SPT * NW
EPT_MLP = E // NW
MCHUNK = -(-EPT_MLP // K)
EPT_MLP_PAD = MCHUNK * K
E_MLP_PAD = EPT_MLP_PAD * NW
AGG_CH = E_MLP_PAD // K

_mesh = plsc.VectorSubcoreMesh(core_axis_name="c", subcore_axis_name="s")


@functools.partial(
    pl.kernel,
    out_type=jax.ShapeDtypeStruct((NPAD * D,), jnp.float32),
    mesh=_mesh,
    scratch_types=[
        pltpu.VMEM((2, K), jnp.int32),
        pltpu.VMEM((2, K), jnp.int32),
        pltpu.VMEM((2, K, D), jnp.float32),
        pltpu.VMEM(((SPT + 1) * D,), jnp.float32),
        pltpu.VMEM((16,), jnp.int32),
        pltpu.SemaphoreType.DMA,
        pltpu.SemaphoreType.DMA,
    ],
)
def _sc_aggregate(y_hbm, src_hbm, dst_hbm, bounds_hbm, out_hbm,
                  sidx, didx, rows, acc, bnd, sem0, sem1):
    cid = lax.axis_index("c")
    sid = lax.axis_index("s")
    wid = cid * NS + sid
    lo = wid * SPT

    pltpu.sync_copy(bounds_hbm.at[wid], bnd)

    def zrow(r, _):
        acc[pl.ds(16 * r, 16)] = jnp.zeros((16,), jnp.float32)
        return 0
    lax.fori_loop(0, SPT * D // 16, zrow, 0)

    bv = bnd[...]
    c0 = bv[0]
    c1 = bv[1]

    NV = D // 16
    zero16 = jnp.zeros((16,), jnp.float32)
    sems = (sem0, sem1)

    def clamp(c):
        return jnp.where(c < c1, c, AGG_CH - 1)

    def issue(c, b):
        pltpu.sync_copy(src_hbm.at[c], sidx.at[b])
        pltpu.sync_copy(dst_hbm.at[c], didx.at[b])
        pltpu.async_copy(y_hbm.at[sidx.at[b]], rows.at[b], sems[b])

    def wait(b):
        pltpu.make_async_copy(y_hbm.at[sidx.at[b]], rows.at[b], sems[b]).wait()

    def compute(b, carry):
        def group(g, carry):
            rprev, racc = carry
            dvec = didx[b, pl.ds(16 * g, 16)]
            for k in range(16):
                p = 16 * g + k
                r = dvec[k] - lo
                new_run = r != rprev
                racc = [
                    jnp.where(new_run, zero16, racc[v])
                    + rows[b, p, pl.ds(16 * v, 16)]
                    for v in range(NV)
                ]
                rd = jnp.where((r >= 0) & (r < SPT), r, SPT) * D
                for v in range(NV):
                    acc[pl.ds(rd + 16 * v, 16)] = racc[v]
                rprev = r
            return rprev, racc
        return lax.fori_loop(0, K // 16, group, carry)

    issue(clamp(c0), 0)
    npairs = (c1 - c0 + 1) // 2

    def pair(i, carry):
        ca = c0 + 2 * i
        wait(0)
        issue(clamp(ca + 1), 1)
        carry = compute(0, carry)
        wait(1)
        issue(clamp(ca + 2), 0)
        carry = compute(1, carry)
        return carry

    lax.fori_loop(0, npairs, pair, (jnp.int32(-1), [zero16] * NV))
    wait(0)

    pltpu.sync_copy(acc.at[pl.ds(0, SPT * D)],
                    out_hbm.at[pl.ds(lo * D, SPT * D)])


@functools.partial(
    pl.kernel,
    out_type=jax.ShapeDtypeStruct((E_MLP_PAD,), jnp.float32),
    mesh=_mesh,
    scratch_types=[
        pltpu.VMEM((K,), jnp.int32),
        pltpu.VMEM((K,), jnp.int32),
        pltpu.VMEM((K, D), jnp.float32),
        pltpu.VMEM((K, D), jnp.float32),
        pltpu.VMEM((K,), jnp.float32),
        pltpu.VMEM((D,), jnp.float32),
        pltpu.VMEM((16,), jnp.float32),
        pltpu.SemaphoreType.DMA,
        pltpu.SemaphoreType.DMA,
    ],
)
def _sc_edge_mlp(a_hbm, b_hbm, src_hbm, dst_hbm, w2_hbm, b2_hbm, out_hbm,
                 sidx, didx, rows_a, rows_b, outv, w2v, b2v, sem_a, sem_b):
    cid = lax.axis_index("c")
    sid = lax.axis_index("s")
    wid = cid * NS + sid

    pltpu.sync_copy(w2_hbm, w2v)
    pltpu.sync_copy(b2_hbm, b2v)

    w2 = [w2v[pl.ds(v * 16, 16)] for v in range(D // 16)]
    b2 = b2v[...]
    lanes = lax.iota(jnp.int32, 16)
    perms = [jnp.bitwise_xor(lanes, off) for off in (8, 4, 2, 1)]

    def _lanesum(x):
        for p in perms:
            x = x + x.at[p].get(mode="promise_in_bounds")
        return x

    def chunk(j, _):
        c = wid * MCHUNK + j
        pltpu.sync_copy(src_hbm.at[c], sidx)
        pltpu.sync_copy(dst_hbm.at[c], didx)
        ca = pltpu.async_copy(a_hbm.at[sidx], rows_a, sem_a)
        cb = pltpu.async_copy(b_hbm.at[didx], rows_b, sem_b)
        ca.wait()
        cb.wait()

        def group(g, _):
            base = g * 16
            ovec = b2
            for k in range(16):
                r = base + k
                acc = jnp.zeros((16,), jnp.float32)
                for v in range(D // 16):
                    s = (rows_a[r, pl.ds(v * 16, 16)]
                         + rows_b[r, pl.ds(v * 16, 16)])
                    acc = acc + jnp.maximum(s, 0.0) * w2[v]
                ovec = jnp.where(lanes == k, _lanesum(acc), ovec)
            outv[pl.ds(base, 16)] = ovec
            return 0
        lax.fori_loop(0, K // 16, group, 0)

        pltpu.sync_copy(outv, out_hbm.at[pl.ds((wid * MCHUNK + j) * K, K)])
        return 0
    lax.fori_loop(0, MCHUNK, chunk, 0)


_RB = 1000
_GRID = N // _RB

_prec = None


def _tc_pre_body(x_ref, w0_ref, b0_ref, deg_ref, x0_ref, y0_ref, dinv_ref):
    dinv = 1.0 / jnp.sqrt(deg_ref[...] + 1.0)
    h = jnp.maximum(jnp.dot(x_ref[...], w0_ref[...], precision=_prec)
                    + b0_ref[...], 0.0)
    x0_ref[...] = h
    y0_ref[...] = dinv * h
    dinv_ref[...] = dinv


def _tc_pre(x, w0, b0, deg):
    return pl.pallas_call(
        _tc_pre_body,
        grid=(_GRID,),
        in_specs=[
            pl.BlockSpec((_RB, D), lambda i: (i, 0)),
            pl.BlockSpec((D, D), lambda i: (0, 0)),
            pl.BlockSpec((1, D), lambda i: (0, 0)),
            pl.BlockSpec((_RB, 1), lambda i: (i, 0)),
        ],
        out_specs=[
            pl.BlockSpec((_RB, D), lambda i: (i, 0)),
            pl.BlockSpec((_RB, D), lambda i: (i, 0)),
            pl.BlockSpec((_RB, 1), lambda i: (i, 0)),
        ],
        out_shape=[
            jax.ShapeDtypeStruct((N, D), jnp.float32),
            jax.ShapeDtypeStruct((N, D), jnp.float32),
            jax.ShapeDtypeStruct((N, 1), jnp.float32),
        ],
    )(x, w0, b0, deg)


def _tc_layer_body(beta, p_ref, y_ref, x0_ref, dinv_ref, w_ref,
                   xl_ref, ynew_ref):
    dinv = dinv_ref[...]
    agg = dinv * (p_ref[...] + y_ref[...])
    xi = (1.0 - ALPHA) * agg + ALPHA * x0_ref[...]
    t = jnp.dot(xi, w_ref[...], precision=_prec)
    xl = jnp.maximum((1.0 - beta) * xi + beta * t, 0.0)
    xl_ref[...] = xl
    ynew_ref[...] = dinv * xl


def _tc_layer(beta, partial, y, x0, dinv, w):
    return pl.pallas_call(
        functools.partial(_tc_layer_body, beta),
        grid=(_GRID,),
        in_specs=[
            pl.BlockSpec((_RB, D), lambda i: (i, 0)),
            pl.BlockSpec((_RB, D), lambda i: (i, 0)),
            pl.BlockSpec((_RB, D), lambda i: (i, 0)),
            pl.BlockSpec((_RB, 1), lambda i: (i, 0)),
            pl.BlockSpec((D, D), lambda i: (0, 0)),
        ],
        out_specs=[
            pl.BlockSpec((_RB, D), lambda i: (i, 0)),
            pl.BlockSpec((_RB, D), lambda i: (i, 0)),
        ],
        out_shape=[
            jax.ShapeDtypeStruct((N, D), jnp.float32),
            jax.ShapeDtypeStruct((N, D), jnp.float32),
        ],
    )(partial, y, x0, dinv, w)


def _tc_final_body(xl_ref, w1_ref, b1_ref, a_ref, b_ref):
    xl = xl_ref[...]
    a_ref[...] = jnp.dot(xl, w1_ref[0:D], precision=_prec) + b1_ref[...]
    b_ref[...] = jnp.dot(xl, w1_ref[D:2 * D], precision=_prec)


def _tc_final(xl, w1, b1):
    return pl.pallas_call(
        _tc_final_body,
        grid=(_GRID,),
        in_specs=[
            pl.BlockSpec((_RB, D), lambda i: (i, 0)),
            pl.BlockSpec((2 * D, D), lambda i: (0, 0)),
            pl.BlockSpec((1, D), lambda i: (0, 0)),
        ],
        out_specs=[
            pl.BlockSpec((_RB, D), lambda i: (i, 0)),
            pl.BlockSpec((_RB, D), lambda i: (i, 0)),
        ],
        out_shape=[
            jax.ShapeDtypeStruct((N, D), jnp.float32),
            jax.ShapeDtypeStruct((N, D), jnp.float32),
        ],
    )(xl, w1, b1)


def kernel(x, edge_index, W0, b0, Wl, mW1, mb1, mW2, mb2):
    src0 = edge_index[0]
    dst0 = edge_index[1]
    pad = E_MLP_PAD - E

    perm = jnp.argsort(dst0)
    srcs = jnp.concatenate(
        [src0[perm], jnp.zeros((pad,), jnp.int32)]).reshape(AGG_CH, K)
    dsts_flat = jnp.concatenate(
        [dst0[perm], jnp.full((pad,), NPAD, jnp.int32)])
    dsts = dsts_flat.reshape(AGG_CH, K)
    offn = jnp.searchsorted(dsts_flat, jnp.arange(N + 1, dtype=jnp.int32))
    deg = (offn[1:] - offn[:-1]).astype(jnp.float32).reshape(N, 1)
    sb = jnp.minimum(jnp.arange(NW + 1, dtype=jnp.int32) * SPT, N)
    boff = offn[sb]
    cb_lo = boff[:-1] // K
    cb_hi = (boff[1:] + K - 1) // K
    bounds = jnp.pad(jnp.stack([cb_lo, cb_hi], axis=1).astype(jnp.int32),
                     ((0, 0), (0, 14)))

    src_g = jnp.concatenate(
        [src0, jnp.zeros((pad,), jnp.int32)]).reshape(NW * MCHUNK, K)
    dst_g = jnp.concatenate(
        [dst0, jnp.zeros((pad,), jnp.int32)]).reshape(NW * MCHUNK, K)

    x0, y, dinv = _tc_pre(x, W0, b0.reshape(1, D), deg)

    xl = x0
    for l in range(L):
        part = _sc_aggregate(y, srcs, dsts, bounds).reshape(NPAD, D)
        beta = math.log(THETA / (l + 1) + 1.0)
        xl, y = _tc_layer(beta, part[:N], y, x0, dinv, Wl[l])

    a, b = _tc_final(xl, mW1, mb1.reshape(1, D))

    w2 = mW2.reshape(D)
    b2 = jnp.full((16,), mb2[0], jnp.float32)
    out = _sc_edge_mlp(a, b, src_g, dst_g, w2, b2)
    return out[:E].reshape(E, 1)


# speedup vs baseline: 1.2325x; 1.2325x over previous
import functools
import math

import jax
import jax.numpy as jnp
from jax import lax
from jax.experimental import pallas as pl
from jax.experimental.pallas import tpu as pltpu
from jax.experimental.pallas import tpu_sc as plsc

N = 10000
E = 320000
D = 128
L = 8
ALPHA = 0.1
THETA = 0.5

NC = 2
NS = 16
NW = NC * NS
K = 128
SPT = 320
NPAD = SPT * NW
EPT_MLP = E // NW
MCHUNK = -(-EPT_MLP // K)
EPT_MLP_PAD = MCHUNK * K
E_MLP_PAD = EPT_MLP_PAD * NW
AGG_CH = E_MLP_PAD // K

_mesh = plsc.VectorSubcoreMesh(core_axis_name="c", subcore_axis_name="s")


@functools.partial(
    pl.kernel,
    out_type=jax.ShapeDtypeStruct((NPAD * D,), jnp.float32),
    mesh=_mesh,
    scratch_types=[
        pltpu.VMEM((K,), jnp.int32),
        pltpu.VMEM((K,), jnp.int32),
        pltpu.VMEM((K, D), jnp.float32),
        pltpu.VMEM(((SPT + 1) * D,), jnp.float32),
        pltpu.VMEM((16,), jnp.int32),
        pltpu.SemaphoreType.DMA,
    ],
)
def _sc_aggregate(y_hbm, src_hbm, dst_hbm, bounds_hbm, out_hbm,
                  sidx, didx, rows, acc, bnd, sem):
    cid = lax.axis_index("c")
    sid = lax.axis_index("s")
    wid = cid * NS + sid
    lo = wid * SPT

    pltpu.sync_copy(bounds_hbm.at[wid], bnd)

    def zrow(r, _):
        acc[pl.ds(16 * r, 16)] = jnp.zeros((16,), jnp.float32)
        return 0
    lax.fori_loop(0, SPT * D // 16, zrow, 0)

    bv = bnd[...]
    c0 = bv[0]
    c1 = bv[1]

    NV = D // 16
    zero16 = jnp.zeros((16,), jnp.float32)

    def chunk(c, carry):
        rprev, racc = carry
        pltpu.sync_copy(src_hbm.at[c], sidx)
        pltpu.sync_copy(dst_hbm.at[c], didx)
        pltpu.async_copy(y_hbm.at[sidx], rows, sem).wait()

        def group(g, carry):
            rprev, racc = carry
            dvec = didx[pl.ds(16 * g, 16)]
            for k in range(16):
                p = 16 * g + k
                r = dvec[k] - lo
                new_run = r != rprev
                racc = [
                    jnp.where(new_run, zero16, racc[v])
                    + rows[p, pl.ds(16 * v, 16)]
                    for v in range(NV)
                ]
                rd = jnp.where((r >= 0) & (r < SPT), r, SPT) * D
                for v in range(NV):
                    acc[pl.ds(rd + 16 * v, 16)] = racc[v]
                rprev = r
            return rprev, racc
        return lax.fori_loop(0, K // 16, group, (rprev, racc))

    lax.fori_loop(c0, c1, chunk, (jnp.int32(-1), [zero16] * NV))

    pltpu.sync_copy(acc.at[pl.ds(0, SPT * D)],
                    out_hbm.at[pl.ds(lo * D, SPT * D)])


@functools.partial(
    pl.kernel,
    out_type=jax.ShapeDtypeStruct((E_MLP_PAD,), jnp.float32),
    mesh=_mesh,
    scratch_types=[
        pltpu.VMEM((K,), jnp.int32),
        pltpu.VMEM((K,), jnp.int32),
        pltpu.VMEM((K, D), jnp.float32),
        pltpu.VMEM((K, D), jnp.float32),
        pltpu.VMEM((K,), jnp.float32),
        pltpu.VMEM((D,), jnp.float32),
        pltpu.VMEM((16,), jnp.float32),
        pltpu.SemaphoreType.DMA,
        pltpu.SemaphoreType.DMA,
    ],
)
def _sc_edge_mlp(a_hbm, b_hbm, src_hbm, dst_hbm, w2_hbm, b2_hbm, out_hbm,
                 sidx, didx, rows_a, rows_b, outv, w2v, b2v, sem_a, sem_b):
    cid = lax.axis_index("c")
    sid = lax.axis_index("s")
    wid = cid * NS + sid

    pltpu.sync_copy(w2_hbm, w2v)
    pltpu.sync_copy(b2_hbm, b2v)

    w2 = [w2v[pl.ds(v * 16, 16)] for v in range(D // 16)]
    b2 = b2v[...]
    lanes = lax.iota(jnp.int32, 16)
    perms = [jnp.bitwise_xor(lanes, off) for off in (8, 4, 2, 1)]

    def _lanesum(x):
        for p in perms:
            x = x + x.at[p].get(mode="promise_in_bounds")
        return x

    def chunk(j, _):
        c = wid * MCHUNK + j
        pltpu.sync_copy(src_hbm.at[c], sidx)
        pltpu.sync_copy(dst_hbm.at[c], didx)
        ca = pltpu.async_copy(a_hbm.at[sidx], rows_a, sem_a)
        cb = pltpu.async_copy(b_hbm.at[didx], rows_b, sem_b)
        ca.wait()
        cb.wait()

        def group(g, _):
            base = g * 16
            ovec = b2
            for k in range(16):
                r = base + k
                acc = jnp.zeros((16,), jnp.float32)
                for v in range(D // 16):
                    s = (rows_a[r, pl.ds(v * 16, 16)]
                         + rows_b[r, pl.ds(v * 16, 16)])
                    acc = acc + jnp.maximum(s, 0.0) * w2[v]
                ovec = jnp.where(lanes == k, _lanesum(acc), ovec)
            outv[pl.ds(base, 16)] = ovec
            return 0
        lax.fori_loop(0, K // 16, group, 0)

        pltpu.sync_copy(outv, out_hbm.at[pl.ds((wid * MCHUNK + j) * K, K)])
        return 0
    lax.fori_loop(0, MCHUNK, chunk, 0)


_RB = 1000
_GRID = N // _RB

_prec = None


def _tc_pre_body(x_ref, w0_ref, b0_ref, deg_ref, x0_ref, y0_ref, dinv_ref):
    dinv = 1.0 / jnp.sqrt(deg_ref[...] + 1.0)
    h = jnp.maximum(jnp.dot(x_ref[...], w0_ref[...], precision=_prec)
                    + b0_ref[...], 0.0)
    x0_ref[...] = h
    y0_ref[...] = dinv * h
    dinv_ref[...] = dinv


def _tc_pre(x, w0, b0, deg):
    return pl.pallas_call(
        _tc_pre_body,
        grid=(_GRID,),
        in_specs=[
            pl.BlockSpec((_RB, D), lambda i: (i, 0)),
            pl.BlockSpec((D, D), lambda i: (0, 0)),
            pl.BlockSpec((1, D), lambda i: (0, 0)),
            pl.BlockSpec((_RB, 1), lambda i: (i, 0)),
        ],
        out_specs=[
            pl.BlockSpec((_RB, D), lambda i: (i, 0)),
            pl.BlockSpec((_RB, D), lambda i: (i, 0)),
            pl.BlockSpec((_RB, 1), lambda i: (i, 0)),
        ],
        out_shape=[
            jax.ShapeDtypeStruct((N, D), jnp.float32),
            jax.ShapeDtypeStruct((N, D), jnp.float32),
            jax.ShapeDtypeStruct((N, 1), jnp.float32),
        ],
    )(x, w0, b0, deg)


def _tc_layer_body(beta, p_ref, y_ref, x0_ref, dinv_ref, w_ref,
                   xl_ref, ynew_ref):
    dinv = dinv_ref[...]
    agg = dinv * (p_ref[...] + y_ref[...])
    xi = (1.0 - ALPHA) * agg + ALPHA * x0_ref[...]
    t = jnp.dot(xi, w_ref[...], precision=_prec)
    xl = jnp.maximum((1.0 - beta) * xi + beta * t, 0.0)
    xl_ref[...] = xl
    ynew_ref[...] = dinv * xl


def _tc_layer(beta, partial, y, x0, dinv, w):
    return pl.pallas_call(
        functools.partial(_tc_layer_body, beta),
        grid=(_GRID,),
        in_specs=[
            pl.BlockSpec((_RB, D), lambda i: (i, 0)),
            pl.BlockSpec((_RB, D), lambda i: (i, 0)),
            pl.BlockSpec((_RB, D), lambda i: (i, 0)),
            pl.BlockSpec((_RB, 1), lambda i: (i, 0)),
            pl.BlockSpec((D, D), lambda i: (0, 0)),
        ],
        out_specs=[
            pl.BlockSpec((_RB, D), lambda i: (i, 0)),
            pl.BlockSpec((_RB, D), lambda i: (i, 0)),
        ],
        out_shape=[
            jax.ShapeDtypeStruct((N, D), jnp.float32),
            jax.ShapeDtypeStruct((N, D), jnp.float32),
        ],
    )(partial, y, x0, dinv, w)


def _tc_final_body(xl_ref, w1_ref, b1_ref, a_ref, b_ref):
    xl = xl_ref[...]
    a_ref[...] = jnp.dot(xl, w1_ref[0:D], precision=_prec) + b1_ref[...]
    b_ref[...] = jnp.dot(xl, w1_ref[D:2 * D], precision=_prec)


def _tc_final(xl, w1, b1):
    return pl.pallas_call(
        _tc_final_body,
        grid=(_GRID,),
        in_specs=[
            pl.BlockSpec((_RB, D), lambda i: (i, 0)),
            pl.BlockSpec((2 * D, D), lambda i: (0, 0)),
            pl.BlockSpec((1, D), lambda i: (0, 0)),
        ],
        out_specs=[
            pl.BlockSpec((_RB, D), lambda i: (i, 0)),
            pl.BlockSpec((_RB, D), lambda i: (i, 0)),
        ],
        out_shape=[
            jax.ShapeDtypeStruct((N, D), jnp.float32),
            jax.ShapeDtypeStruct((N, D), jnp.float32),
        ],
    )(xl, w1, b1)


def kernel(x, edge_index, W0, b0, Wl, mW1, mb1, mW2, mb2):
    src0 = edge_index[0]
    dst0 = edge_index[1]
    pad = E_MLP_PAD - E

    perm = jnp.argsort(dst0)
    srcs = jnp.concatenate(
        [src0[perm], jnp.zeros((pad,), jnp.int32)]).reshape(AGG_CH, K)
    dsts_flat = jnp.concatenate(
        [dst0[perm], jnp.full((pad,), NPAD, jnp.int32)])
    dsts = dsts_flat.reshape(AGG_CH, K)
    offn = jnp.searchsorted(dsts_flat, jnp.arange(N + 1, dtype=jnp.int32))
    deg = (offn[1:] - offn[:-1]).astype(jnp.float32).reshape(N, 1)
    sb = jnp.minimum(jnp.arange(NW + 1, dtype=jnp.int32) * SPT, N)
    boff = offn[sb]
    cb_lo = boff[:-1] // K
    cb_hi = (boff[1:] + K - 1) // K
    bounds = jnp.pad(jnp.stack([cb_lo, cb_hi], axis=1).astype(jnp.int32),
                     ((0, 0), (0, 14)))

    src_g = jnp.concatenate(
        [src0, jnp.zeros((pad,), jnp.int32)]).reshape(NW * MCHUNK, K)
    dst_g = jnp.concatenate(
        [dst0, jnp.zeros((pad,), jnp.int32)]).reshape(NW * MCHUNK, K)

    x0, y, dinv = _tc_pre(x, W0, b0.reshape(1, D), deg)

    xl = x0
    for l in range(L):
        part = _sc_aggregate(y, srcs, dsts, bounds).reshape(NPAD, D)
        beta = math.log(THETA / (l + 1) + 1.0)
        xl, y = _tc_layer(beta, part[:N], y, x0, dinv, Wl[l])

    a, b = _tc_final(xl, mW1, mb1.reshape(1, D))

    w2 = mW2.reshape(D)
    b2 = jnp.full((16,), mb2[0], jnp.float32)
    out = _sc_edge_mlp(a, b, src_g, dst_g, w2, b2)
    return out[:E].reshape(E, 1)
